# Initial kernel scaffold; baseline (speedup 1.0000x reference)
import jax
import jax.numpy as jnp
from jax.experimental import pallas as pl


def kernel(x, edge_index, batch, W1, b1, W2, b2, W3, b3, c1w, c1b, c2w, c2b, f1w, f1b, f2w, f2b):
    raise NotImplementedError("write your pallas kernel here")


# speedup vs baseline: 15.6740x; 15.6740x over previous
import functools

import jax
import jax.numpy as jnp
from jax import lax
from jax.experimental import pallas as pl
from jax.experimental.pallas import tpu as pltpu
from jax.experimental.pallas import tpu_sc as plsc

_N = 10000
_E = 320000
_B = 4
_K = 2910
_S = 16384
_WN = 40
_NW = 250
_RPS = _N // 16
_R = 1000


def _make_segfeat():
  mesh = plsc.VectorSubcoreMesh(core_axis_name="c", subcore_axis_name="s")
  nw = _E // (16 * _WN)

  @functools.partial(
      pl.kernel,
      out_type=jax.ShapeDtypeStruct((2, _N, 64), jnp.float32),
      mesh=mesh,
      compiler_params=pltpu.CompilerParams(use_tc_tiling_on_sc=False),
      scratch_types=[
          pltpu.VMEM_SHARED((_N, 64), jnp.float32),
          pltpu.VMEM((nw, _WN), jnp.int32),
          pltpu.VMEM((nw, _WN), jnp.int32),
          pltpu.VMEM((_WN, 64), jnp.float32),
          pltpu.VMEM((_WN, 64), jnp.float32),
          pltpu.SemaphoreType.DMA,
          pltpu.SemaphoreType.DMA,
          pltpu.SemaphoreType.DMA,
          pltpu.SemaphoreType.DMA,
      ],
  )
  def seg(u_hbm, row_hbm, col_hbm, zero_hbm, out_hbm,
          acc, rows, cols, buf0, buf1, g0, g1, s0, s1):
    c = lax.axis_index("c")
    s = lax.axis_index("s")
    tab = u_hbm.at[c]
    pltpu.sync_copy(row_hbm.at[s], rows)
    pltpu.sync_copy(col_hbm.at[s], cols)

    @pl.when(s < 15)
    def _():
      pltpu.sync_copy(zero_hbm.at[pl.ds(s * 632, 632)],
                      acc.at[pl.ds(s * 632, 632)])

    @pl.when(s == 15)
    def _():
      pltpu.sync_copy(zero_hbm.at[pl.ds(9480, 520)],
                      acc.at[pl.ds(9480, 520)])

    bufs = (buf0, buf1)
    gsems = (g0, g1)
    ssems = (s0, s1)
    pltpu.async_copy(tab.at[rows.at[0]], buf0, g0)
    pltpu.async_copy(tab.at[rows.at[1]], buf1, g1)
    plsc.subcore_barrier()

    def body(j, carry):
      for b in range(2):
        wdw = j * 2 + b
        pltpu.make_async_copy(tab.at[rows.at[wdw]], bufs[b], gsems[b]).wait()
        pltpu.async_copy(bufs[b], acc.at[cols.at[wdw]], ssems[b],
                         add=True).wait()

        @pl.when(wdw + 2 < nw)
        def _():
          pltpu.async_copy(tab.at[rows.at[wdw + 2]], bufs[b], gsems[b])
      return carry

    lax.fori_loop(0, nw // 2, body, 0)
    plsc.subcore_barrier()

    @pl.when(s < 15)
    def _():
      pltpu.sync_copy(acc.at[pl.ds(s * 632, 632)],
                      out_hbm.at[c].at[pl.ds(s * 632, 632)])

    @pl.when(s == 15)
    def _():
      pltpu.sync_copy(acc.at[pl.ds(9480, 520)],
                      out_hbm.at[c].at[pl.ds(9480, 520)])

  return seg


def _make_segsum(F):
  mesh = plsc.VectorSubcoreMesh(core_axis_name="c", subcore_axis_name="s")

  @functools.partial(
      pl.kernel,
      out_type=jax.ShapeDtypeStruct((2, _N, F), jnp.float32),
      mesh=mesh,
      compiler_params=pltpu.CompilerParams(
          use_tc_tiling_on_sc=(F == 128)),
      scratch_types=[
          pltpu.VMEM_SHARED((_N, F), jnp.float32),
          pltpu.VMEM((_NW, _WN), jnp.int32),
          pltpu.VMEM((_NW, _WN), jnp.int32),
          pltpu.VMEM((_WN, F), jnp.float32),
          pltpu.VMEM((_WN, F), jnp.float32),
          pltpu.SemaphoreType.DMA,
          pltpu.SemaphoreType.DMA,
          pltpu.SemaphoreType.DMA,
          pltpu.SemaphoreType.DMA,
      ],
  )
  def seg(u_hbm, row_hbm, col_hbm, zero_hbm, out_hbm,
          acc, rows, cols, buf0, buf1, g0, g1, s0, s1):
    c = lax.axis_index("c")
    s = lax.axis_index("s")
    w = c * 16 + s
    pltpu.sync_copy(row_hbm.at[w], rows)
    pltpu.sync_copy(col_hbm.at[w], cols)
    @pl.when(s < 15)
    def _():
      pltpu.sync_copy(zero_hbm.at[pl.ds(s * 632, 632)],
                      acc.at[pl.ds(s * 632, 632)])

    @pl.when(s == 15)
    def _():
      pltpu.sync_copy(zero_hbm.at[pl.ds(9480, 520)],
                      acc.at[pl.ds(9480, 520)])
    bufs = (buf0, buf1)
    gsems = (g0, g1)
    ssems = (s0, s1)
    pltpu.async_copy(u_hbm.at[rows.at[0]], buf0, g0)
    pltpu.async_copy(u_hbm.at[rows.at[1]], buf1, g1)
    plsc.subcore_barrier()

    def body(j, carry):
      for b in range(2):
        wdw = j * 2 + b
        pltpu.make_async_copy(u_hbm.at[rows.at[wdw]], bufs[b], gsems[b]).wait()
        pltpu.async_copy(bufs[b], acc.at[cols.at[wdw]], ssems[b],
                         add=True).wait()

        @pl.when(wdw + 2 < _NW)
        def _():
          pltpu.async_copy(u_hbm.at[rows.at[wdw + 2]], bufs[b], gsems[b])
      return carry

    lax.fori_loop(0, _NW // 2, body, 0)
    plsc.subcore_barrier()

    @pl.when(s < 15)
    def _():
      pltpu.sync_copy(acc.at[pl.ds(s * 632, 632)],
                      out_hbm.at[c].at[pl.ds(s * 632, 632)])

    @pl.when(s == 15)
    def _():
      pltpu.sync_copy(acc.at[pl.ds(9480, 520)],
                      out_hbm.at[c].at[pl.ds(9480, 520)])

  return seg


_HALF = pl.BlockSpec((1, _R, 64), lambda i: (0, i, 0))
_HALF2 = pl.BlockSpec((1, _R, 64), lambda i: (1, i, 0))


def _split_store(o_ref, u):
  o_ref[0, :, :] = u[:, :64]
  o_ref[1, :, :] = u[:, 64:]


def _dense1(x, w1t, b1, dega, degb):
  def body(x_ref, w_ref, b_ref, da_ref, db_ref, u_ref, dis_ref):
    deg = da_ref[:, :1] + db_ref[:, :1] + 1.0
    dis = lax.rsqrt(deg)
    t = jnp.tanh(jnp.dot(x_ref[...], w_ref[...],
                         preferred_element_type=jnp.float32) + b_ref[...])
    _split_store(u_ref, dis * t)
    dis_ref[...] = jnp.broadcast_to(dis, (_R, 8))

  return pl.pallas_call(
      body,
      grid=(_N // _R,),
      in_specs=[
          pl.BlockSpec((_R, 128), lambda i: (i, 0)),
          pl.BlockSpec((128, 128), lambda i: (0, 0)),
          pl.BlockSpec((1, 128), lambda i: (0, 0)),
          pl.BlockSpec((_R, 8), lambda i: (i, 0)),
          pl.BlockSpec((_R, 8), lambda i: (i, 0)),
      ],
      out_specs=[
          pl.BlockSpec((2, _R, 64), lambda i: (0, i, 0)),
          pl.BlockSpec((_R, 8), lambda i: (i, 0)),
      ],
      out_shape=[
          jax.ShapeDtypeStruct((2, _N, 64), jnp.float32),
          jax.ShapeDtypeStruct((_N, 8), jnp.float32),
      ],
  )(x, w1t, b1, dega, degb)


def _dense_next(ss, uu, dis8, wt, b, fout):
  def body(sl_ref, sr_ref, ul_ref, ur_ref, d_ref, w_ref, b_ref, u_ref):
    dis = d_ref[:, :1]
    zl = dis * (sl_ref[0] + ul_ref[0])
    zr = dis * (sr_ref[0] + ur_ref[0])
    t = jnp.tanh(jnp.dot(zl, w_ref[:64, :], preferred_element_type=jnp.float32)
                 + jnp.dot(zr, w_ref[64:, :], preferred_element_type=jnp.float32)
                 + b_ref[...])
    u = dis * t
    if fout == 128:
      _split_store(u_ref, u)
    else:
      u_ref[...] = u

  if fout == 128:
    out_spec = pl.BlockSpec((2, _R, 64), lambda i: (0, i, 0))
    out_shape = jax.ShapeDtypeStruct((2, _N, 64), jnp.float32)
  else:
    out_spec = pl.BlockSpec((_R, fout), lambda i: (i, 0))
    out_shape = jax.ShapeDtypeStruct((_N, fout), jnp.float32)
  return pl.pallas_call(
      body,
      grid=(_N // _R,),
      in_specs=[
          _HALF, _HALF2, _HALF, _HALF2,
          pl.BlockSpec((_R, 8), lambda i: (i, 0)),
          pl.BlockSpec((128, fout), lambda i: (0, 0)),
          pl.BlockSpec((1, fout), lambda i: (0, 0)),
      ],
      out_specs=out_spec,
      out_shape=out_shape,
  )(ss, ss, uu, uu, dis8, wt, b)


def _combine3(p0, p1, u3, dis8):
  def body(p0_ref, p1_ref, u_ref, d_ref, v_ref):
    v_ref[...] = d_ref[...] * (p0_ref[...] + p1_ref[...] + u_ref[...])

  spec = pl.BlockSpec((_R, 8), lambda i: (i, 0))
  return pl.pallas_call(
      body,
      grid=(_N // _R,),
      in_specs=[spec, spec, spec, spec],
      out_specs=spec,
      out_shape=jax.ShapeDtypeStruct((_N, 8), jnp.float32),
  )(p0, p1, u3, dis8)


def _sortpool(vpad, bpad):
  def body(v_ref, b_ref, o_ref):
    neg = jnp.float32(-jnp.inf)
    bt = b_ref[...]
    m = jnp.where(
        bt[None, :, :] == lax.broadcasted_iota(jnp.int32, (_B, 128, 128), 0),
        v_ref[...][None, :, :], neg)
    m = m.reshape(_B * 128, 128)
    r_i = lax.broadcasted_iota(jnp.int32, (_B * 128, 128), 0) & 127
    c_i = lax.broadcasted_iota(jnp.int32, (_B * 128, 128), 1)
    idx = r_i * 128 + c_i
    k = 2
    while k <= _S:
      j = k // 2
      while j >= 1:
        if j >= 128:
          jr = j // 128
          up = pltpu.roll(m, (_B * 128) - jr, axis=0)
          dn = pltpu.roll(m, jr, axis=0)
          part = jnp.where((r_i & jr) == 0, up, dn)
        else:
          up = pltpu.roll(m, 128 - j, axis=1)
          dn = pltpu.roll(m, j, axis=1)
          part = jnp.where((c_i & j) == 0, up, dn)
        is_lo = (idx & j) == 0
        descblk = (idx & k) == 0
        m = jnp.where(descblk == is_lo,
                      jnp.maximum(m, part), jnp.minimum(m, part))
        j //= 2
      k *= 2
    m3 = m.reshape(_B, 128, 128)
    top = m3[:, :23, :]
    o_ref[...] = jnp.where(top > neg, top, 0.0)

  return pl.pallas_call(
      body,
      out_shape=jax.ShapeDtypeStruct((_B, 23, 128), jnp.float32),
  )(vpad, bpad)


def _head(pw, c1wt, c1b, c2r, c2b, f1r, f1b, f2wt, f2b):
  def body(pw_ref, c1_ref, c1b_ref, c2_ref, c2b_ref, f1_ref, f1b_ref,
           f2_ref, f2b_ref, o_ref):
    y1 = jnp.maximum(
        jnp.dot(pw_ref[...], c1_ref[...],
                preferred_element_type=jnp.float32) + c1b_ref[...], 0.0)
    mp = jnp.max(y1.reshape(60, 2, 16), axis=1)
    mm = mp.reshape(_B, 15, 16)
    acc = jnp.zeros((_B * 11, 32), jnp.float32)
    for kk in range(5):
      wnd = mm[:, kk:kk + 11, :].reshape(_B * 11, 16)
      acc = acc + jnp.dot(wnd, c2_ref[kk],
                          preferred_element_type=jnp.float32)
    y3 = jnp.maximum(acc + c2b_ref[...], 0.0).reshape(_B, 11, 32)
    acc2 = jnp.zeros((_B, 128), jnp.float32)
    for ss in range(11):
      acc2 = acc2 + jnp.dot(y3[:, ss, :], f1_ref[ss],
                            preferred_element_type=jnp.float32)
    f = jnp.maximum(acc2 + f1b_ref[...], 0.0)
    o_ref[...] = jnp.dot(f, f2_ref[...],
                         preferred_element_type=jnp.float32) + f2b_ref[...]

  return pl.pallas_call(
      body,
      out_shape=jax.ShapeDtypeStruct((_B, 10), jnp.float32),
  )(pw, c1wt, c1b, c2r, c2b, f1r, f1b, f2wt, f2b)


def kernel(x, edge_index, batch, W1, b1, W2, b2, W3, b3,
           c1w, c1b, c2w, c2b, f1w, f1b, f2w, f2b):
  f32 = jnp.float32
  row3 = edge_index[0].reshape(32, _NW, _WN)
  col3 = edge_index[1].reshape(32, _NW, _WN)
  row16 = edge_index[0].reshape(16, _E // (16 * _WN), _WN)
  col16 = edge_index[1].reshape(16, _E // (16 * _WN), _WN)
  zeros64 = jnp.zeros((_N, 64), f32)
  zeros8 = jnp.zeros((_N, 8), f32)
  ones8 = jnp.ones((_N, 8), f32)

  segf = _make_segfeat()
  seg8 = _make_segsum(8)

  degp = seg8(ones8, row3, col3, zeros8)

  u1, dis8 = _dense1(x, W1.T, b1.reshape(1, 128), degp[0], degp[1])
  s1 = segf(u1, row16, col16, zeros64)
  u2 = _dense_next(s1, u1, dis8, W2.T, b2.reshape(1, 128), 128)
  s2 = segf(u2, row16, col16, zeros64)
  w3t = jnp.zeros((128, 8), f32).at[:, :1].set(W3.T)
  b3p = jnp.zeros((1, 8), f32).at[0, :1].set(b3)
  u3 = _dense_next(s2, u2, dis8, w3t, b3p, 8)
  p3 = seg8(u3, row3, col3, zeros8)
  v = _combine3(p3[0], p3[1], u3, dis8)

  vpad = jnp.concatenate(
      [v[:, 0], jnp.full((_S - _N,), -jnp.inf, f32)]).reshape(128, 128)
  bpad = jnp.concatenate(
      [batch, jnp.full((_S - _N,), -1, jnp.int32)]).reshape(128, 128)
  top = _sortpool(vpad, bpad)
  p = top.reshape(_B, 23 * 128)[:, :_K]

  pw = p.reshape(_B * 30, 97)
  c1wt = c1w[:, 0, :].T
  c2r = jnp.transpose(c2w, (2, 1, 0))
  f1r = jnp.transpose(f1w.reshape(128, 32, 11), (2, 1, 0))
  return _head(pw, c1wt, c1b.reshape(1, 16), c2r, c2b.reshape(1, 32),
               f1r, f1b.reshape(1, 128), f2w.T, f2b.reshape(1, 10))


# speedup vs baseline: 24.1947x; 1.5436x over previous
import functools

import jax
import jax.numpy as jnp
from jax import lax
from jax.experimental import pallas as pl
from jax.experimental.pallas import tpu as pltpu
from jax.experimental.pallas import tpu_sc as plsc

_N = 10000
_E = 320000
_B = 4
_K = 2910
_S = 16384
_WN = 100
_NW = 100
_RPS = _N // 16
_R = 1000


def _make_seg(F, feat_split):
  mesh = plsc.VectorSubcoreMesh(core_axis_name="c", subcore_axis_name="s")
  nworker = 16 if feat_split else 32
  nw = _E // (nworker * _WN)
  tab_shape = (2, _N, F) if feat_split else (_N, F)

  @functools.partial(
      pl.kernel,
      out_type=jax.ShapeDtypeStruct((2, _N, F), jnp.float32),
      mesh=mesh,
      compiler_params=pltpu.CompilerParams(use_tc_tiling_on_sc=False),
      scratch_types=[
          pltpu.VMEM_SHARED((_N, F), jnp.float32),
          pltpu.VMEM((nw, _WN), jnp.int32),
          pltpu.VMEM((nw, _WN), jnp.int32),
          pltpu.VMEM((_WN, F), jnp.float32),
          pltpu.VMEM((_WN, F), jnp.float32),
          pltpu.VMEM((_WN, F), jnp.float32),
          pltpu.VMEM((_WN, F), jnp.float32),
          pltpu.SemaphoreType.DMA,
          pltpu.SemaphoreType.DMA,
          pltpu.SemaphoreType.DMA,
          pltpu.SemaphoreType.DMA,
          pltpu.SemaphoreType.DMA,
          pltpu.SemaphoreType.DMA,
          pltpu.SemaphoreType.DMA,
          pltpu.SemaphoreType.DMA,
      ],
  )
  def seg(u_hbm, row_hbm, col_hbm, zero_hbm, out_hbm,
          acc, rows, cols, b0, b1, b2, b3,
          g0, g1, g2, g3, s0, s1, s2, s3):
    c = lax.axis_index("c")
    s = lax.axis_index("s")
    tab = u_hbm.at[c] if feat_split else u_hbm
    widx = s if feat_split else c * 16 + s
    pltpu.sync_copy(row_hbm.at[widx], rows)
    pltpu.sync_copy(col_hbm.at[widx], cols)

    @pl.when(s < 15)
    def _():
      pltpu.sync_copy(zero_hbm.at[pl.ds(s * 632, 632)],
                      acc.at[pl.ds(s * 632, 632)])

    @pl.when(s == 15)
    def _():
      pltpu.sync_copy(zero_hbm.at[pl.ds(9480, 520)],
                      acc.at[pl.ds(9480, 520)])

    bufs = (b0, b1, b2, b3)
    gsems = (g0, g1, g2, g3)
    ssems = (s0, s1, s2, s3)
    pltpu.async_copy(tab.at[rows.at[0]], b0, g0)
    pltpu.async_copy(tab.at[rows.at[1]], b1, g1)
    plsc.subcore_barrier()

    def body(j, carry):
      for b in range(4):
        w = j * 4 + b
        pltpu.make_async_copy(tab.at[rows.at[w]], bufs[b], gsems[b]).wait()
        pltpu.async_copy(bufs[b], acc.at[cols.at[w]], ssems[b], add=True)
        nb = (b + 2) % 4

        @pl.when(w >= 2)
        def _():
          pltpu.make_async_copy(tab.at[pl.ds(0, _WN)], bufs[nb],
                                ssems[nb]).wait()

        @pl.when(w + 2 < nw)
        def _():
          pltpu.async_copy(tab.at[rows.at[w + 2]], bufs[nb], gsems[nb])
      return carry

    lax.fori_loop(0, nw // 4, body, 0)
    pltpu.make_async_copy(tab.at[pl.ds(0, _WN)], b0, ssems[(nw - 2) % 4]).wait()
    pltpu.make_async_copy(tab.at[pl.ds(0, _WN)], b1, ssems[(nw - 1) % 4]).wait()
    plsc.subcore_barrier()

    @pl.when(s < 15)
    def _():
      pltpu.sync_copy(acc.at[pl.ds(s * 632, 632)],
                      out_hbm.at[c].at[pl.ds(s * 632, 632)])

    @pl.when(s == 15)
    def _():
      pltpu.sync_copy(acc.at[pl.ds(9480, 520)],
                      out_hbm.at[c].at[pl.ds(9480, 520)])

  return seg


_HALF = pl.BlockSpec((1, _R, 64), lambda i: (0, i, 0))
_HALF2 = pl.BlockSpec((1, _R, 64), lambda i: (1, i, 0))


def _split_store(o_ref, u):
  o_ref[0, :, :] = u[:, :64]
  o_ref[1, :, :] = u[:, 64:]


def _dense1(x, w1t, b1, dega, degb):
  def body(x_ref, w_ref, b_ref, da_ref, db_ref, u_ref, dis_ref):
    deg = da_ref[:, :1] + db_ref[:, :1] + 1.0
    dis = lax.rsqrt(deg)
    t = jnp.tanh(jnp.dot(x_ref[...], w_ref[...],
                         preferred_element_type=jnp.float32) + b_ref[...])
    _split_store(u_ref, dis * t)
    dis_ref[...] = jnp.broadcast_to(dis, (_R, 8))

  return pl.pallas_call(
      body,
      grid=(_N // _R,),
      in_specs=[
          pl.BlockSpec((_R, 128), lambda i: (i, 0)),
          pl.BlockSpec((128, 128), lambda i: (0, 0)),
          pl.BlockSpec((1, 128), lambda i: (0, 0)),
          pl.BlockSpec((_R, 8), lambda i: (i, 0)),
          pl.BlockSpec((_R, 8), lambda i: (i, 0)),
      ],
      out_specs=[
          pl.BlockSpec((2, _R, 64), lambda i: (0, i, 0)),
          pl.BlockSpec((_R, 8), lambda i: (i, 0)),
      ],
      out_shape=[
          jax.ShapeDtypeStruct((2, _N, 64), jnp.float32),
          jax.ShapeDtypeStruct((_N, 8), jnp.float32),
      ],
  )(x, w1t, b1, dega, degb)


def _dense_next(ss, uu, dis8, wt, b, fout):
  def body(sl_ref, sr_ref, ul_ref, ur_ref, d_ref, w_ref, b_ref, u_ref):
    dis = d_ref[:, :1]
    zl = dis * (sl_ref[0] + ul_ref[0])
    zr = dis * (sr_ref[0] + ur_ref[0])
    t = jnp.tanh(jnp.dot(zl, w_ref[:64, :], preferred_element_type=jnp.float32)
                 + jnp.dot(zr, w_ref[64:, :], preferred_element_type=jnp.float32)
                 + b_ref[...])
    u = dis * t
    if fout == 128:
      _split_store(u_ref, u)
    else:
      u_ref[...] = u

  if fout == 128:
    out_spec = pl.BlockSpec((2, _R, 64), lambda i: (0, i, 0))
    out_shape = jax.ShapeDtypeStruct((2, _N, 64), jnp.float32)
  else:
    out_spec = pl.BlockSpec((_R, fout), lambda i: (i, 0))
    out_shape = jax.ShapeDtypeStruct((_N, fout), jnp.float32)
  return pl.pallas_call(
      body,
      grid=(_N // _R,),
      in_specs=[
          _HALF, _HALF2, _HALF, _HALF2,
          pl.BlockSpec((_R, 8), lambda i: (i, 0)),
          pl.BlockSpec((128, fout), lambda i: (0, 0)),
          pl.BlockSpec((1, fout), lambda i: (0, 0)),
      ],
      out_specs=out_spec,
      out_shape=out_shape,
  )(ss, ss, uu, uu, dis8, wt, b)


def _combine3(p0, p1, u3, dis8):
  def body(p0_ref, p1_ref, u_ref, d_ref, v_ref):
    v_ref[...] = d_ref[...] * (p0_ref[...] + p1_ref[...] + u_ref[...])

  spec = pl.BlockSpec((_R, 8), lambda i: (i, 0))
  return pl.pallas_call(
      body,
      grid=(_N // _R,),
      in_specs=[spec, spec, spec, spec],
      out_specs=spec,
      out_shape=jax.ShapeDtypeStruct((_N, 8), jnp.float32),
  )(p0, p1, u3, dis8)


def _sortpool(vpad, bpad):
  def body(v_ref, b_ref, o_ref):
    neg = jnp.float32(-jnp.inf)
    bt = b_ref[...]
    m = jnp.where(
        bt[None, :, :] == lax.broadcasted_iota(jnp.int32, (_B, 128, 128), 0),
        v_ref[...][None, :, :], neg)
    m = m.reshape(_B * 128, 128)
    r_i = lax.broadcasted_iota(jnp.int32, (_B * 128, 128), 0) & 127
    c_i = lax.broadcasted_iota(jnp.int32, (_B * 128, 128), 1)
    idx = r_i * 128 + c_i
    k = 2
    while k <= _S:
      j = k // 2
      while j >= 1:
        if j >= 128:
          jr = j // 128
          up = pltpu.roll(m, (_B * 128) - jr, axis=0)
          dn = pltpu.roll(m, jr, axis=0)
          part = jnp.where((r_i & jr) == 0, up, dn)
        else:
          up = pltpu.roll(m, 128 - j, axis=1)
          dn = pltpu.roll(m, j, axis=1)
          part = jnp.where((c_i & j) == 0, up, dn)
        is_lo = (idx & j) == 0
        descblk = (idx & k) == 0
        m = jnp.where(descblk == is_lo,
                      jnp.maximum(m, part), jnp.minimum(m, part))
        j //= 2
      k *= 2
    m3 = m.reshape(_B, 128, 128)
    top = m3[:, :23, :]
    o_ref[...] = jnp.where(top > neg, top, 0.0)

  return pl.pallas_call(
      body,
      out_shape=jax.ShapeDtypeStruct((_B, 23, 128), jnp.float32),
  )(vpad, bpad)


def _head(pw, c1wt, c1b, c2r, c2b, f1r, f1b, f2wt, f2b):
  def body(pw_ref, c1_ref, c1b_ref, c2_ref, c2b_ref, f1_ref, f1b_ref,
           f2_ref, f2b_ref, o_ref):
    y1 = jnp.maximum(
        jnp.dot(pw_ref[...], c1_ref[...],
                preferred_element_type=jnp.float32) + c1b_ref[...], 0.0)
    mp = jnp.max(y1.reshape(60, 2, 16), axis=1)
    mm = mp.reshape(_B, 15, 16)
    acc = jnp.zeros((_B * 11, 32), jnp.float32)
    for kk in range(5):
      wnd = mm[:, kk:kk + 11, :].reshape(_B * 11, 16)
      acc = acc + jnp.dot(wnd, c2_ref[kk],
                          preferred_element_type=jnp.float32)
    y3 = jnp.maximum(acc + c2b_ref[...], 0.0).reshape(_B, 11, 32)
    acc2 = jnp.zeros((_B, 128), jnp.float32)
    for ss in range(11):
      acc2 = acc2 + jnp.dot(y3[:, ss, :], f1_ref[ss],
                            preferred_element_type=jnp.float32)
    f = jnp.maximum(acc2 + f1b_ref[...], 0.0)
    o_ref[...] = jnp.dot(f, f2_ref[...],
                         preferred_element_type=jnp.float32) + f2b_ref[...]

  return pl.pallas_call(
      body,
      out_shape=jax.ShapeDtypeStruct((_B, 10), jnp.float32),
  )(pw, c1wt, c1b, c2r, c2b, f1r, f1b, f2wt, f2b)


def kernel(x, edge_index, batch, W1, b1, W2, b2, W3, b3,
           c1w, c1b, c2w, c2b, f1w, f1b, f2w, f2b):
  f32 = jnp.float32
  row3 = edge_index[0].reshape(32, _NW, _WN)
  col3 = edge_index[1].reshape(32, _NW, _WN)
  row16 = edge_index[0].reshape(16, _E // (16 * _WN), _WN)
  col16 = edge_index[1].reshape(16, _E // (16 * _WN), _WN)
  zeros64 = jnp.zeros((_N, 64), f32)
  zeros8 = jnp.zeros((_N, 8), f32)
  ones8 = jnp.ones((_N, 8), f32)

  segf = _make_seg(64, True)
  seg8 = _make_seg(8, False)

  degp = seg8(ones8, row3, col3, zeros8)

  u1, dis8 = _dense1(x, W1.T, b1.reshape(1, 128), degp[0], degp[1])
  s1 = segf(u1, row16, col16, zeros64)
  u2 = _dense_next(s1, u1, dis8, W2.T, b2.reshape(1, 128), 128)
  s2 = segf(u2, row16, col16, zeros64)
  w3t = jnp.zeros((128, 8), f32).at[:, :1].set(W3.T)
  b3p = jnp.zeros((1, 8), f32).at[0, :1].set(b3)
  u3 = _dense_next(s2, u2, dis8, w3t, b3p, 8)
  p3 = seg8(u3, row3, col3, zeros8)
  v = _combine3(p3[0], p3[1], u3, dis8)

  vpad = jnp.concatenate(
      [v[:, 0], jnp.full((_S - _N,), -jnp.inf, f32)]).reshape(128, 128)
  bpad = jnp.concatenate(
      [batch, jnp.full((_S - _N,), -1, jnp.int32)]).reshape(128, 128)
  top = _sortpool(vpad, bpad)
  p = top.reshape(_B, 23 * 128)[:, :_K]

  pw = p.reshape(_B * 30, 97)
  c1wt = c1w[:, 0, :].T
  c2r = jnp.transpose(c2w, (2, 1, 0))
  f1r = jnp.transpose(f1w.reshape(128, 32, 11), (2, 1, 0))
  return _head(pw, c1wt, c1b.reshape(1, 16), c2r, c2b.reshape(1, 32),
               f1r, f1b.reshape(1, 128), f2w.T, f2b.reshape(1, 10))


# speedup vs baseline: 25.5318x; 1.0553x over previous
import functools

import jax
import jax.numpy as jnp
from jax import lax
from jax.experimental import pallas as pl
from jax.experimental.pallas import tpu as pltpu
from jax.experimental.pallas import tpu_sc as plsc

_N = 10000
_E = 320000
_B = 4
_K = 2910
_S = 16384
_WN = 100
_NW = 100
_RPS = _N // 16
_R = 1000


def _make_seg(F, feat_split, gather=True):
  mesh = plsc.VectorSubcoreMesh(core_axis_name="c", subcore_axis_name="s")
  nworker = 16 if feat_split else 32
  nw = _E // (nworker * _WN)
  tab_shape = (2, _N, F) if feat_split else (_N, F)

  @functools.partial(
      pl.kernel,
      out_type=jax.ShapeDtypeStruct((2, _N, F), jnp.float32),
      mesh=mesh,
      compiler_params=pltpu.CompilerParams(use_tc_tiling_on_sc=False),
      scratch_types=[
          pltpu.VMEM_SHARED((_N, F), jnp.float32),
          pltpu.VMEM((nw, _WN), jnp.int32),
          pltpu.VMEM((nw, _WN), jnp.int32),
          pltpu.VMEM((_WN, F), jnp.float32),
          pltpu.VMEM((_WN, F), jnp.float32),
          pltpu.VMEM((_WN, F), jnp.float32),
          pltpu.VMEM((_WN, F), jnp.float32),
          pltpu.SemaphoreType.DMA,
          pltpu.SemaphoreType.DMA,
          pltpu.SemaphoreType.DMA,
          pltpu.SemaphoreType.DMA,
          pltpu.SemaphoreType.DMA,
          pltpu.SemaphoreType.DMA,
          pltpu.SemaphoreType.DMA,
          pltpu.SemaphoreType.DMA,
      ],
  )
  def seg(u_hbm, row_hbm, col_hbm, zero_hbm, out_hbm,
          acc, rows, cols, b0, b1, b2, b3,
          g0, g1, g2, g3, s0, s1, s2, s3):
    c = lax.axis_index("c")
    s = lax.axis_index("s")
    tab = u_hbm.at[c] if feat_split else u_hbm
    widx = s if feat_split else c * 16 + s
    pltpu.sync_copy(row_hbm.at[widx], rows)
    pltpu.sync_copy(col_hbm.at[widx], cols)

    @pl.when(s < 15)
    def _():
      pltpu.sync_copy(zero_hbm.at[pl.ds(s * 632, 632)],
                      acc.at[pl.ds(s * 632, 632)])

    @pl.when(s == 15)
    def _():
      pltpu.sync_copy(zero_hbm.at[pl.ds(9480, 520)],
                      acc.at[pl.ds(9480, 520)])

    bufs = (b0, b1, b2, b3)
    gsems = (g0, g1, g2, g3)
    ssems = (s0, s1, s2, s3)
    if gather:
      pltpu.async_copy(tab.at[rows.at[0]], b0, g0)
      pltpu.async_copy(tab.at[rows.at[1]], b1, g1)
    else:
      for _b in bufs:
        pltpu.sync_copy(tab.at[pl.ds(0, _WN)], _b)
    plsc.subcore_barrier()

    def body(j, carry):
      for b in range(4):
        w = j * 4 + b
        if gather:
          pltpu.make_async_copy(tab.at[rows.at[w]], bufs[b], gsems[b]).wait()
        pltpu.async_copy(bufs[b], acc.at[cols.at[w]], ssems[b], add=True)
        nb = (b + 2) % 4

        @pl.when(w >= 2)
        def _():
          pltpu.make_async_copy(tab.at[pl.ds(0, _WN)], bufs[nb],
                                ssems[nb]).wait()

        if gather:
          @pl.when(w + 2 < nw)
          def _():
            pltpu.async_copy(tab.at[rows.at[w + 2]], bufs[nb], gsems[nb])
      return carry

    lax.fori_loop(0, nw // 4, body, 0)
    pltpu.make_async_copy(tab.at[pl.ds(0, _WN)], b0, ssems[(nw - 2) % 4]).wait()
    pltpu.make_async_copy(tab.at[pl.ds(0, _WN)], b1, ssems[(nw - 1) % 4]).wait()
    plsc.subcore_barrier()

    @pl.when(s < 15)
    def _():
      pltpu.sync_copy(acc.at[pl.ds(s * 632, 632)],
                      out_hbm.at[c].at[pl.ds(s * 632, 632)])

    @pl.when(s == 15)
    def _():
      pltpu.sync_copy(acc.at[pl.ds(9480, 520)],
                      out_hbm.at[c].at[pl.ds(9480, 520)])

  return seg


_HALF = pl.BlockSpec((1, _R, 64), lambda i: (0, i, 0))
_HALF2 = pl.BlockSpec((1, _R, 64), lambda i: (1, i, 0))


def _split_store(o_ref, u):
  o_ref[0, :, :] = u[:, :64]
  o_ref[1, :, :] = u[:, 64:]


def _dense1(x, w1t, b1, dega, degb):
  def body(x_ref, w_ref, b_ref, da_ref, db_ref, u_ref, dis_ref):
    deg = da_ref[:, :1] + db_ref[:, :1] + 1.0
    dis = lax.rsqrt(deg)
    t = jnp.tanh(jnp.dot(x_ref[...], w_ref[...],
                         preferred_element_type=jnp.float32) + b_ref[...])
    _split_store(u_ref, dis * t)
    dis_ref[...] = jnp.broadcast_to(dis, (_R, 8))

  return pl.pallas_call(
      body,
      grid=(_N // _R,),
      in_specs=[
          pl.BlockSpec((_R, 128), lambda i: (i, 0)),
          pl.BlockSpec((128, 128), lambda i: (0, 0)),
          pl.BlockSpec((1, 128), lambda i: (0, 0)),
          pl.BlockSpec((_R, 8), lambda i: (i, 0)),
          pl.BlockSpec((_R, 8), lambda i: (i, 0)),
      ],
      out_specs=[
          pl.BlockSpec((2, _R, 64), lambda i: (0, i, 0)),
          pl.BlockSpec((_R, 8), lambda i: (i, 0)),
      ],
      out_shape=[
          jax.ShapeDtypeStruct((2, _N, 64), jnp.float32),
          jax.ShapeDtypeStruct((_N, 8), jnp.float32),
      ],
  )(x, w1t, b1, dega, degb)


def _dense_next(ss, uu, dis8, wt, b, fout):
  def body(sl_ref, sr_ref, ul_ref, ur_ref, d_ref, w_ref, b_ref, u_ref):
    dis = d_ref[:, :1]
    zl = dis * (sl_ref[0] + ul_ref[0])
    zr = dis * (sr_ref[0] + ur_ref[0])
    t = jnp.tanh(jnp.dot(zl, w_ref[:64, :], preferred_element_type=jnp.float32)
                 + jnp.dot(zr, w_ref[64:, :], preferred_element_type=jnp.float32)
                 + b_ref[...])
    u = dis * t
    if fout == 128:
      _split_store(u_ref, u)
    else:
      u_ref[...] = u

  if fout == 128:
    out_spec = pl.BlockSpec((2, _R, 64), lambda i: (0, i, 0))
    out_shape = jax.ShapeDtypeStruct((2, _N, 64), jnp.float32)
  else:
    out_spec = pl.BlockSpec((_R, fout), lambda i: (i, 0))
    out_shape = jax.ShapeDtypeStruct((_N, fout), jnp.float32)
  return pl.pallas_call(
      body,
      grid=(_N // _R,),
      in_specs=[
          _HALF, _HALF2, _HALF, _HALF2,
          pl.BlockSpec((_R, 8), lambda i: (i, 0)),
          pl.BlockSpec((128, fout), lambda i: (0, 0)),
          pl.BlockSpec((1, fout), lambda i: (0, 0)),
      ],
      out_specs=out_spec,
      out_shape=out_shape,
  )(ss, ss, uu, uu, dis8, wt, b)


def _combine3(p0, p1, u3, dis8):
  def body(p0_ref, p1_ref, u_ref, d_ref, v_ref):
    v_ref[...] = d_ref[...] * (p0_ref[...] + p1_ref[...] + u_ref[...])

  spec = pl.BlockSpec((_R, 8), lambda i: (i, 0))
  return pl.pallas_call(
      body,
      grid=(_N // _R,),
      in_specs=[spec, spec, spec, spec],
      out_specs=spec,
      out_shape=jax.ShapeDtypeStruct((_N, 8), jnp.float32),
  )(p0, p1, u3, dis8)


def _sortpool(vpad, bpad):
  def body(v_ref, b_ref, o_ref):
    neg = jnp.float32(-jnp.inf)
    bt = b_ref[...]
    m = jnp.where(
        bt[None, :, :] == lax.broadcasted_iota(jnp.int32, (_B, 128, 128), 0),
        v_ref[...][None, :, :], neg)
    m = m.reshape(_B * 128, 128)
    r_i = lax.broadcasted_iota(jnp.int32, (_B * 128, 128), 0) & 127
    c_i = lax.broadcasted_iota(jnp.int32, (_B * 128, 128), 1)
    idx = r_i * 128 + c_i
    k = 2
    while k <= _S:
      j = k // 2
      while j >= 1:
        if j >= 128:
          jr = j // 128
          up = pltpu.roll(m, (_B * 128) - jr, axis=0)
          dn = pltpu.roll(m, jr, axis=0)
          part = jnp.where((r_i & jr) == 0, up, dn)
        else:
          up = pltpu.roll(m, 128 - j, axis=1)
          dn = pltpu.roll(m, j, axis=1)
          part = jnp.where((c_i & j) == 0, up, dn)
        is_lo = (idx & j) == 0
        descblk = (idx & k) == 0
        m = jnp.where(descblk == is_lo,
                      jnp.maximum(m, part), jnp.minimum(m, part))
        j //= 2
      k *= 2
    m3 = m.reshape(_B, 128, 128)
    top = m3[:, :23, :]
    o_ref[...] = jnp.where(top > neg, top, 0.0)

  return pl.pallas_call(
      body,
      out_shape=jax.ShapeDtypeStruct((_B, 23, 128), jnp.float32),
  )(vpad, bpad)


def _head(pw, c1wt, c1b, c2r, c2b, f1r, f1b, f2wt, f2b):
  def body(pw_ref, c1_ref, c1b_ref, c2_ref, c2b_ref, f1_ref, f1b_ref,
           f2_ref, f2b_ref, o_ref):
    y1 = jnp.maximum(
        jnp.dot(pw_ref[...], c1_ref[...],
                preferred_element_type=jnp.float32) + c1b_ref[...], 0.0)
    mp = jnp.max(y1.reshape(60, 2, 16), axis=1)
    mm = mp.reshape(_B, 15, 16)
    acc = jnp.zeros((_B * 11, 32), jnp.float32)
    for kk in range(5):
      wnd = mm[:, kk:kk + 11, :].reshape(_B * 11, 16)
      acc = acc + jnp.dot(wnd, c2_ref[kk],
                          preferred_element_type=jnp.float32)
    y3 = jnp.maximum(acc + c2b_ref[...], 0.0).reshape(_B, 11, 32)
    acc2 = jnp.zeros((_B, 128), jnp.float32)
    for ss in range(11):
      acc2 = acc2 + jnp.dot(y3[:, ss, :], f1_ref[ss],
                            preferred_element_type=jnp.float32)
    f = jnp.maximum(acc2 + f1b_ref[...], 0.0)
    o_ref[...] = jnp.dot(f, f2_ref[...],
                         preferred_element_type=jnp.float32) + f2b_ref[...]

  return pl.pallas_call(
      body,
      out_shape=jax.ShapeDtypeStruct((_B, 10), jnp.float32),
  )(pw, c1wt, c1b, c2r, c2b, f1r, f1b, f2wt, f2b)


def kernel(x, edge_index, batch, W1, b1, W2, b2, W3, b3,
           c1w, c1b, c2w, c2b, f1w, f1b, f2w, f2b):
  f32 = jnp.float32
  row3 = edge_index[0].reshape(32, _NW, _WN)
  col3 = edge_index[1].reshape(32, _NW, _WN)
  row16 = edge_index[0].reshape(16, _E // (16 * _WN), _WN)
  col16 = edge_index[1].reshape(16, _E // (16 * _WN), _WN)
  zeros64 = jnp.zeros((_N, 64), f32)
  zeros8 = jnp.zeros((_N, 8), f32)
  ones8 = jnp.ones((_N, 8), f32)

  segf = _make_seg(64, True)
  seg8 = _make_seg(8, False)

  deg_seg = _make_seg(8, False, gather=False)
  degp = deg_seg(ones8, row3, col3, zeros8)

  u1, dis8 = _dense1(x, W1.T, b1.reshape(1, 128), degp[0], degp[1])
  s1 = segf(u1, row16, col16, zeros64)
  u2 = _dense_next(s1, u1, dis8, W2.T, b2.reshape(1, 128), 128)
  s2 = segf(u2, row16, col16, zeros64)
  w3t = jnp.zeros((128, 8), f32).at[:, :1].set(W3.T)
  b3p = jnp.zeros((1, 8), f32).at[0, :1].set(b3)
  u3 = _dense_next(s2, u2, dis8, w3t, b3p, 8)
  p3 = seg8(u3, row3, col3, zeros8)
  v = _combine3(p3[0], p3[1], u3, dis8)

  vpad = jnp.concatenate(
      [v[:, 0], jnp.full((_S - _N,), -jnp.inf, f32)]).reshape(128, 128)
  bpad = jnp.concatenate(
      [batch, jnp.full((_S - _N,), -1, jnp.int32)]).reshape(128, 128)
  top = _sortpool(vpad, bpad)
  p = top.reshape(_B, 23 * 128)[:, :_K]

  pw = p.reshape(_B * 30, 97)
  c1wt = c1w[:, 0, :].T
  c2r = jnp.transpose(c2w, (2, 1, 0))
  f1r = jnp.transpose(f1w.reshape(128, 32, 11), (2, 1, 0))
  return _head(pw, c1wt, c1b.reshape(1, 16), c2r, c2b.reshape(1, 32),
               f1r, f1b.reshape(1, 128), f2w.T, f2b.reshape(1, 10))


# speedup vs baseline: 26.8738x; 1.0526x over previous
import functools

import jax
import jax.numpy as jnp
from jax import lax
from jax.experimental import pallas as pl
from jax.experimental.pallas import tpu as pltpu
from jax.experimental.pallas import tpu_sc as plsc

_N = 10000
_E = 320000
_B = 4
_K = 2910
_S = 16384
_WN = 100
_NW = 100
_RPS = _N // 16
_R = 1000


def _make_seg(F, feat_split, gather=True):
  mesh = plsc.VectorSubcoreMesh(core_axis_name="c", subcore_axis_name="s")
  nworker = 16 if feat_split else 32
  nw = _E // (nworker * _WN)
  oneD = F == 1
  out_t = (2, _N) if oneD else (2, _N, F)
  buf_t = (_WN,) if oneD else (_WN, F)

  @functools.partial(
      pl.kernel,
      out_type=jax.ShapeDtypeStruct(out_t, jnp.float32),
      mesh=mesh,
      compiler_params=pltpu.CompilerParams(use_tc_tiling_on_sc=False),
      scratch_types=[
          pltpu.VMEM_SHARED((_N,) if oneD else (_N, F), jnp.float32),
          pltpu.VMEM((nw, _WN), jnp.int32),
          pltpu.VMEM((nw, _WN), jnp.int32),
          pltpu.VMEM(buf_t, jnp.float32),
          pltpu.VMEM(buf_t, jnp.float32),
          pltpu.VMEM(buf_t, jnp.float32),
          pltpu.VMEM(buf_t, jnp.float32),
          pltpu.SemaphoreType.DMA,
          pltpu.SemaphoreType.DMA,
          pltpu.SemaphoreType.DMA,
          pltpu.SemaphoreType.DMA,
          pltpu.SemaphoreType.DMA,
          pltpu.SemaphoreType.DMA,
          pltpu.SemaphoreType.DMA,
          pltpu.SemaphoreType.DMA,
      ],
  )
  def seg(u_hbm, row_hbm, col_hbm, zero_hbm, out_hbm,
          acc, rows, cols, b0, b1, b2, b3,
          g0, g1, g2, g3, s0, s1, s2, s3):
    c = lax.axis_index("c")
    s = lax.axis_index("s")
    tab = u_hbm.at[c] if feat_split else u_hbm
    if feat_split:
      pltpu.sync_copy(row_hbm.at[s], rows)
      pltpu.sync_copy(col_hbm.at[s], cols)
    else:
      pltpu.sync_copy(row_hbm.at[s, pl.ds(c * nw, nw)], rows)
      pltpu.sync_copy(col_hbm.at[s, pl.ds(c * nw, nw)], cols)

    @pl.when(s < 15)
    def _():
      pltpu.sync_copy(zero_hbm.at[pl.ds(s * 632, 632)],
                      acc.at[pl.ds(s * 632, 632)])

    @pl.when(s == 15)
    def _():
      pltpu.sync_copy(zero_hbm.at[pl.ds(9480, 520)],
                      acc.at[pl.ds(9480, 520)])

    bufs = (b0, b1, b2, b3)
    gsems = (g0, g1, g2, g3)
    ssems = (s0, s1, s2, s3)
    if gather:
      pltpu.async_copy(tab.at[rows.at[0]], b0, g0)
      pltpu.async_copy(tab.at[rows.at[1]], b1, g1)
    else:
      for _b in bufs:
        pltpu.sync_copy(tab.at[pl.ds(0, _WN)], _b)
    plsc.subcore_barrier()

    def body(j, carry):
      for b in range(4):
        w = j * 4 + b
        if gather:
          pltpu.make_async_copy(tab.at[rows.at[w]], bufs[b], gsems[b]).wait()
        pltpu.async_copy(bufs[b], acc.at[cols.at[w]], ssems[b], add=True)
        nb = (b + 2) % 4

        @pl.when(w >= 2)
        def _():
          pltpu.make_async_copy(tab.at[pl.ds(0, _WN)], bufs[nb],
                                ssems[nb]).wait()

        if gather:
          @pl.when(w + 2 < nw)
          def _():
            pltpu.async_copy(tab.at[rows.at[w + 2]], bufs[nb], gsems[nb])
      return carry

    lax.fori_loop(0, nw // 4, body, 0)
    pltpu.make_async_copy(tab.at[pl.ds(0, _WN)], b0, ssems[(nw - 2) % 4]).wait()
    pltpu.make_async_copy(tab.at[pl.ds(0, _WN)], b1, ssems[(nw - 1) % 4]).wait()
    plsc.subcore_barrier()

    @pl.when(s < 15)
    def _():
      pltpu.sync_copy(acc.at[pl.ds(s * 632, 632)],
                      out_hbm.at[c].at[pl.ds(s * 632, 632)])

    @pl.when(s == 15)
    def _():
      pltpu.sync_copy(acc.at[pl.ds(9480, 520)],
                      out_hbm.at[c].at[pl.ds(9480, 520)])

  return seg


_HALF = pl.BlockSpec((1, _R, 64), lambda i: (0, i, 0))
_HALF2 = pl.BlockSpec((1, _R, 64), lambda i: (1, i, 0))


def _split_store(o_ref, u):
  o_ref[0, :, :] = u[:, :64]
  o_ref[1, :, :] = u[:, 64:]


def _dense1(x, w1t, b1, dega, degb):
  def body(x_ref, w_ref, b_ref, da_ref, db_ref, u_ref, dis_ref):
    deg = da_ref[...] + db_ref[...] + 1.0
    dis = lax.rsqrt(deg)
    t = jnp.tanh(jnp.dot(x_ref[...], w_ref[...],
                         preferred_element_type=jnp.float32) + b_ref[...])
    _split_store(u_ref, dis * t)
    dis_ref[...] = jnp.broadcast_to(dis, (_R, 8))

  return pl.pallas_call(
      body,
      grid=(_N // _R,),
      in_specs=[
          pl.BlockSpec((_R, 128), lambda i: (i, 0)),
          pl.BlockSpec((128, 128), lambda i: (0, 0)),
          pl.BlockSpec((1, 128), lambda i: (0, 0)),
          pl.BlockSpec((_R, 1), lambda i: (i, 0)),
          pl.BlockSpec((_R, 1), lambda i: (i, 0)),
      ],
      out_specs=[
          pl.BlockSpec((2, _R, 64), lambda i: (0, i, 0)),
          pl.BlockSpec((_R, 8), lambda i: (i, 0)),
      ],
      out_shape=[
          jax.ShapeDtypeStruct((2, _N, 64), jnp.float32),
          jax.ShapeDtypeStruct((_N, 8), jnp.float32),
      ],
  )(x, w1t, b1, dega, degb)


def _dense_next(ss, uu, dis8, wt, b, fout):
  def body(sl_ref, sr_ref, ul_ref, ur_ref, d_ref, w_ref, b_ref, u_ref):
    dis = d_ref[:, :1]
    zl = dis * (sl_ref[0] + ul_ref[0])
    zr = dis * (sr_ref[0] + ur_ref[0])
    t = jnp.tanh(jnp.dot(zl, w_ref[:64, :], preferred_element_type=jnp.float32)
                 + jnp.dot(zr, w_ref[64:, :], preferred_element_type=jnp.float32)
                 + b_ref[...])
    u = dis * t
    if fout == 128:
      _split_store(u_ref, u)
    else:
      u_ref[...] = u

  if fout == 128:
    out_spec = pl.BlockSpec((2, _R, 64), lambda i: (0, i, 0))
    out_shape = jax.ShapeDtypeStruct((2, _N, 64), jnp.float32)
  else:
    out_spec = pl.BlockSpec((_R, fout), lambda i: (i, 0))
    out_shape = jax.ShapeDtypeStruct((_N, fout), jnp.float32)
  return pl.pallas_call(
      body,
      grid=(_N // _R,),
      in_specs=[
          _HALF, _HALF2, _HALF, _HALF2,
          pl.BlockSpec((_R, 8), lambda i: (i, 0)),
          pl.BlockSpec((128, fout), lambda i: (0, 0)),
          pl.BlockSpec((1, fout), lambda i: (0, 0)),
      ],
      out_specs=out_spec,
      out_shape=out_shape,
  )(ss, ss, uu, uu, dis8, wt, b)


def _combine3(p0, p1, u3, dis8):
  def body(p0_ref, p1_ref, u_ref, d_ref, v_ref):
    v_ref[...] = d_ref[...] * (p0_ref[...] + p1_ref[...] + u_ref[...])

  spec = pl.BlockSpec((_R, 1), lambda i: (i, 0))
  return pl.pallas_call(
      body,
      grid=(_N // _R,),
      in_specs=[spec, spec, spec, spec],
      out_specs=spec,
      out_shape=jax.ShapeDtypeStruct((_N, 1), jnp.float32),
  )(p0, p1, u3, dis8)


def _sortpool(vpad, bpad):
  def body(v_ref, b_ref, o_ref):
    neg = jnp.float32(-jnp.inf)
    bt = b_ref[...]
    m = jnp.where(
        bt[None, :, :] == lax.broadcasted_iota(jnp.int32, (_B, 128, 128), 0),
        v_ref[...][None, :, :], neg)
    m = m.reshape(_B * 128, 128)
    r_i = lax.broadcasted_iota(jnp.int32, (_B * 128, 128), 0) & 127
    c_i = lax.broadcasted_iota(jnp.int32, (_B * 128, 128), 1)
    idx = r_i * 128 + c_i
    k = 2
    while k <= _S:
      j = k // 2
      while j >= 1:
        if j >= 128:
          jr = j // 128
          up = pltpu.roll(m, (_B * 128) - jr, axis=0)
          dn = pltpu.roll(m, jr, axis=0)
          part = jnp.where((r_i & jr) == 0, up, dn)
        else:
          up = pltpu.roll(m, 128 - j, axis=1)
          dn = pltpu.roll(m, j, axis=1)
          part = jnp.where((c_i & j) == 0, up, dn)
        is_lo = (idx & j) == 0
        descblk = (idx & k) == 0
        m = jnp.where(descblk == is_lo,
                      jnp.maximum(m, part), jnp.minimum(m, part))
        j //= 2
      k *= 2
    m3 = m.reshape(_B, 128, 128)
    top = m3[:, :23, :]
    o_ref[...] = jnp.where(top > neg, top, 0.0)

  return pl.pallas_call(
      body,
      out_shape=jax.ShapeDtypeStruct((_B, 23, 128), jnp.float32),
  )(vpad, bpad)


def _head(pw, c1wt, c1b, c2r, c2b, f1r, f1b, f2wt, f2b):
  def body(pw_ref, c1_ref, c1b_ref, c2_ref, c2b_ref, f1_ref, f1b_ref,
           f2_ref, f2b_ref, o_ref):
    y1 = jnp.maximum(
        jnp.dot(pw_ref[...], c1_ref[...],
                preferred_element_type=jnp.float32) + c1b_ref[...], 0.0)
    mp = jnp.max(y1.reshape(60, 2, 16), axis=1)
    mm = mp.reshape(_B, 15, 16)
    acc = jnp.zeros((_B * 11, 32), jnp.float32)
    for kk in range(5):
      wnd = mm[:, kk:kk + 11, :].reshape(_B * 11, 16)
      acc = acc + jnp.dot(wnd, c2_ref[kk],
                          preferred_element_type=jnp.float32)
    y3 = jnp.maximum(acc + c2b_ref[...], 0.0).reshape(_B, 11, 32)
    acc2 = jnp.zeros((_B, 128), jnp.float32)
    for ss in range(11):
      acc2 = acc2 + jnp.dot(y3[:, ss, :], f1_ref[ss],
                            preferred_element_type=jnp.float32)
    f = jnp.maximum(acc2 + f1b_ref[...], 0.0)
    o_ref[...] = jnp.dot(f, f2_ref[...],
                         preferred_element_type=jnp.float32) + f2b_ref[...]

  return pl.pallas_call(
      body,
      out_shape=jax.ShapeDtypeStruct((_B, 10), jnp.float32),
  )(pw, c1wt, c1b, c2r, c2b, f1r, f1b, f2wt, f2b)


def kernel(x, edge_index, batch, W1, b1, W2, b2, W3, b3,
           c1w, c1b, c2w, c2b, f1w, f1b, f2w, f2b):
  f32 = jnp.float32
  row16 = edge_index[0].reshape(16, _E // (16 * _WN), _WN)
  col16 = edge_index[1].reshape(16, _E // (16 * _WN), _WN)
  zeros64 = jnp.zeros((_N, 64), f32)
  zeros1 = jnp.zeros((_N,), f32)
  ones1 = jnp.ones((_N,), f32)

  segf = _make_seg(64, True)
  seg1 = _make_seg(1, False)

  deg_seg = _make_seg(1, False, gather=False)
  degp = deg_seg(ones1, row16, col16, zeros1)

  u1, dis8 = _dense1(x, W1.T, b1.reshape(1, 128),
                     degp[0].reshape(_N, 1), degp[1].reshape(_N, 1))
  s1 = segf(u1, row16, col16, zeros64)
  u2 = _dense_next(s1, u1, dis8, W2.T, b2.reshape(1, 128), 128)
  s2 = segf(u2, row16, col16, zeros64)
  w3t = jnp.zeros((128, 8), f32).at[:, :1].set(W3.T)
  b3p = jnp.zeros((1, 8), f32).at[0, :1].set(b3)
  u3 = _dense_next(s2, u2, dis8, w3t, b3p, 8)
  p3 = seg1(u3[:, 0], row16, col16, zeros1)
  v = _combine3(p3[0].reshape(_N, 1), p3[1].reshape(_N, 1),
                u3[:, :1], dis8[:, :1])

  vpad = jnp.concatenate(
      [v[:, 0], jnp.full((_S - _N,), -jnp.inf, f32)]).reshape(128, 128)
  bpad = jnp.concatenate(
      [batch, jnp.full((_S - _N,), -1, jnp.int32)]).reshape(128, 128)
  top = _sortpool(vpad, bpad)
  p = top.reshape(_B, 23 * 128)[:, :_K]

  pw = p.reshape(_B * 30, 97)
  c1wt = c1w[:, 0, :].T
  c2r = jnp.transpose(c2w, (2, 1, 0))
  f1r = jnp.transpose(f1w.reshape(128, 32, 11), (2, 1, 0))
  return _head(pw, c1wt, c1b.reshape(1, 16), c2r, c2b.reshape(1, 32),
               f1r, f1b.reshape(1, 128), f2w.T, f2b.reshape(1, 10))
